# baseline (speedup 1.0000x reference)
import functools

import jax
import jax.numpy as jnp
from jax import lax
from jax.experimental import pallas as pl
from jax.experimental.pallas import tpu as pltpu
from jax.experimental.pallas import tpu_sc as plsc

EMB_D = 64
BATCH = 16384
NEMB = 1000000
RPD = NEMB // 16
NC = 2
NS = 16
NW = NC * NS
B_W = BATCH // NW
ICH = 128
NIC = B_W // ICH
NK = B_W // 16


def _mf_kernel(ids1_hbm, ids2_hbm, e1f_hbm, e2f_hbm, out_hbm,
               idx1_v, idx2_v, idr1_v, idr2_v, off1_v, off2_v,
               six1_v, six2_v, d1_v, d2_v, acc_v, sem1, sem2):
    wid = lax.axis_index("s") * NC + lax.axis_index("c")

    pltpu.sync_copy(ids1_hbm.at[wid], idx1_v)
    pltpu.sync_copy(ids2_hbm.at[wid], idx2_v)

    lane = lax.iota(jnp.int32, 16)

    def prep(k, _):
        s = pl.ds(k * 16, 16)
        i1 = idx1_v[s]
        i2 = idx2_v[s]
        idr1_v[s] = lax.shift_right_logical(i1, 4)
        idr2_v[s] = lax.shift_right_logical(i2, 4)
        off1_v[s] = jnp.bitwise_and(i1, 15)
        off2_v[s] = jnp.bitwise_and(i2, 15)
        acc_v[s] = jnp.zeros((16,), jnp.float32)
        return 0

    lax.fori_loop(0, NK, prep, 0)

    def issue(d, b):
        base = d * RPD

        def bld(k, _):
            s = pl.ds(k * 16, 16)
            six1_v[b, s] = idr1_v[s] + base
            six2_v[b, s] = idr2_v[s] + base
            return 0

        lax.fori_loop(0, NK, bld, 0)
        for c in range(NIC):
            s = pl.ds(c * ICH, ICH)
            pltpu.async_copy(e1f_hbm.at[six1_v.at[b, s]], d1_v.at[b, s], sem1)
            pltpu.async_copy(e2f_hbm.at[six2_v.at[b, s]], d2_v.at[b, s], sem2)

    def drain(b):
        for c in range(NIC):
            s = pl.ds(c * ICH, ICH)
            pltpu.make_async_copy(
                e1f_hbm.at[six1_v.at[b, s]], d1_v.at[b, s], sem1).wait()
            pltpu.make_async_copy(
                e2f_hbm.at[six2_v.at[b, s]], d2_v.at[b, s], sem2).wait()

    issue(0, 0)
    issue(1, 1)

    def step(d, _):
        b = lax.rem(d, 2)
        bvec = jnp.full((16,), b, jnp.int32)
        drain(b)

        def mac(k, _):
            s = pl.ds(k * 16, 16)
            jvec = lane + (k * 16)
            v1 = plsc.load_gather(d1_v, [bvec, jvec, off1_v[s]])
            v2 = plsc.load_gather(d2_v, [bvec, jvec, off2_v[s]])
            acc_v[s] = acc_v[s] + v1 * v2
            return 0

        lax.fori_loop(0, NK, mac, 0)

        @pl.when(d + 2 < EMB_D)
        def _():
            issue(d + 2, b)

        return 0

    lax.fori_loop(0, EMB_D, step, 0)

    pltpu.sync_copy(acc_v, out_hbm.at[wid])


@jax.jit
def kernel(ids1, ids2, emb1, emb2):
    mesh = plsc.VectorSubcoreMesh(core_axis_name="c", subcore_axis_name="s",
                                  num_cores=NC, num_subcores=NS)
    k = functools.partial(
        pl.kernel,
        out_type=jax.ShapeDtypeStruct((NW, B_W), jnp.float32),
        mesh=mesh,
        compiler_params=pltpu.CompilerParams(use_tc_tiling_on_sc=False,
                                             needs_layout_passes=False),
        scratch_types=[
            pltpu.VMEM((B_W,), jnp.int32),
            pltpu.VMEM((B_W,), jnp.int32),
            pltpu.VMEM((B_W,), jnp.int32),
            pltpu.VMEM((B_W,), jnp.int32),
            pltpu.VMEM((B_W,), jnp.int32),
            pltpu.VMEM((B_W,), jnp.int32),
            pltpu.VMEM((2, B_W), jnp.int32),
            pltpu.VMEM((2, B_W), jnp.int32),
            pltpu.VMEM((2, B_W, 16), jnp.float32),
            pltpu.VMEM((2, B_W, 16), jnp.float32),
            pltpu.VMEM((B_W,), jnp.float32),
            pltpu.SemaphoreType.DMA,
            pltpu.SemaphoreType.DMA,
        ],
    )(_mf_kernel)
    ids1_2d = ids1.astype(jnp.int32).reshape(NW, B_W)
    ids2_2d = ids2.astype(jnp.int32).reshape(NW, B_W)
    e1f = emb1.T.reshape(EMB_D * NEMB // 16, 16)
    e2f = emb2.T.reshape(EMB_D * NEMB // 16, 16)
    out = k(ids1_2d, ids2_2d, e1f, e2f)
    return out.reshape(BATCH, 1)


# speedup vs baseline: 1.0009x; 1.0009x over previous
import functools

import jax
import jax.numpy as jnp
from jax import lax
from jax.experimental import pallas as pl
from jax.experimental.pallas import tpu as pltpu
from jax.experimental.pallas import tpu_sc as plsc

EMB_D = 64
BATCH = 16384
NEMB = 1000000
RPD = NEMB // 16
NC = 2
NS = 16
NW = NC * NS
B_W = BATCH // NW
ICH = 128
NIC = B_W // ICH
NK = B_W // 16


def _mf_kernel(ids1_hbm, ids2_hbm, e1f_hbm, e2f_hbm, out_hbm,
               idx1_v, idx2_v, idr1_v, idr2_v, off1_v, off2_v,
               six1_v, six2_v, d1_v, d2_v, acc_v, sem1, sem2):
    wid = lax.axis_index("s") * NC + lax.axis_index("c")

    pltpu.sync_copy(ids1_hbm.at[wid], idx1_v)
    pltpu.sync_copy(ids2_hbm.at[wid], idx2_v)

    lane = lax.iota(jnp.int32, 16)

    def prep(k, _):
        s = pl.ds(k * 16, 16)
        i1 = idx1_v[s]
        i2 = idx2_v[s]
        idr1_v[s] = lax.shift_right_logical(i1, 4)
        idr2_v[s] = lax.shift_right_logical(i2, 4)
        off1_v[s] = jnp.bitwise_and(i1, 15)
        off2_v[s] = jnp.bitwise_and(i2, 15)
        acc_v[s] = jnp.zeros((16,), jnp.float32)
        return 0

    lax.fori_loop(0, NK, prep, 0)

    def issue(d, b):
        base = d * RPD

        def bld(k, _):
            s = pl.ds(k * 16, 16)
            six1_v[b, s] = idr1_v[s] + base
            six2_v[b, s] = idr2_v[s] + base
            return 0

        lax.fori_loop(0, NK, bld, 0)
        for c in range(NIC):
            s = pl.ds(c * ICH, ICH)
            pltpu.async_copy(e1f_hbm.at[six1_v.at[b, s]], d1_v.at[b, s], sem1)
            pltpu.async_copy(e2f_hbm.at[six2_v.at[b, s]], d2_v.at[b, s], sem2)

    def drain(b):
        for c in range(NIC):
            s = pl.ds(c * ICH, ICH)
            pltpu.make_async_copy(
                e1f_hbm.at[six1_v.at[b, s]], d1_v.at[b, s], sem1).wait()
            pltpu.make_async_copy(
                e2f_hbm.at[six2_v.at[b, s]], d2_v.at[b, s], sem2).wait()

    dbase = jnp.bitwise_and(wid * 2, EMB_D - 1)

    def dmap(i):
        return jnp.bitwise_and(i + dbase, EMB_D - 1)

    issue(dmap(0), 0)
    issue(dmap(1), 1)

    def step(i2, _):
        for b in range(2):
            i = i2 * 2 + b
            bvec = jnp.full((16,), b, jnp.int32)
            drain(b)

            def mac(k, _):
                s = pl.ds(k * 16, 16)
                jvec = lane + (k * 16)
                v1 = plsc.load_gather(d1_v, [bvec, jvec, off1_v[s]])
                v2 = plsc.load_gather(d2_v, [bvec, jvec, off2_v[s]])
                acc_v[s] = acc_v[s] + v1 * v2
                return 0

            lax.fori_loop(0, NK, mac, 0)

            @pl.when(i + 2 < EMB_D)
            def _():
                issue(dmap(i + 2), b)

        return 0

    lax.fori_loop(0, EMB_D // 2, step, 0)

    pltpu.sync_copy(acc_v, out_hbm.at[wid])


@jax.jit
def kernel(ids1, ids2, emb1, emb2):
    mesh = plsc.VectorSubcoreMesh(core_axis_name="c", subcore_axis_name="s",
                                  num_cores=NC, num_subcores=NS)
    k = functools.partial(
        pl.kernel,
        out_type=jax.ShapeDtypeStruct((NW, B_W), jnp.float32),
        mesh=mesh,
        compiler_params=pltpu.CompilerParams(use_tc_tiling_on_sc=False,
                                             needs_layout_passes=False),
        scratch_types=[
            pltpu.VMEM((B_W,), jnp.int32),
            pltpu.VMEM((B_W,), jnp.int32),
            pltpu.VMEM((B_W,), jnp.int32),
            pltpu.VMEM((B_W,), jnp.int32),
            pltpu.VMEM((B_W,), jnp.int32),
            pltpu.VMEM((B_W,), jnp.int32),
            pltpu.VMEM((2, B_W), jnp.int32),
            pltpu.VMEM((2, B_W), jnp.int32),
            pltpu.VMEM((2, B_W, 16), jnp.float32),
            pltpu.VMEM((2, B_W, 16), jnp.float32),
            pltpu.VMEM((B_W,), jnp.float32),
            pltpu.SemaphoreType.DMA,
            pltpu.SemaphoreType.DMA,
        ],
    )(_mf_kernel)
    ids1_2d = ids1.astype(jnp.int32).reshape(NW, B_W)
    ids2_2d = ids2.astype(jnp.int32).reshape(NW, B_W)
    e1f = emb1.T.reshape(EMB_D * NEMB // 16, 16)
    e2f = emb2.T.reshape(EMB_D * NEMB // 16, 16)
    out = k(ids1_2d, ids2_2d, e1f, e2f)
    return out.reshape(BATCH, 1)


# speedup vs baseline: 11.4200x; 11.4102x over previous
import functools

import jax
import jax.numpy as jnp
from jax import lax
from jax.experimental import pallas as pl
from jax.experimental.pallas import tpu as pltpu
from jax.experimental.pallas import tpu_sc as plsc

EMB_D = 64
BATCH = 16384
NC = 2
NS = 16
NW = NC * NS
B_W = BATCH // NW
CH = 128
NCHUNK = B_W // CH
ICH = 128


def _permute(v, idx16):
    dnums = lax.GatherDimensionNumbers(
        offset_dims=(), collapsed_slice_dims=(0,), start_index_map=(0,))
    return lax.gather(v, idx16[:, None], dnums, slice_sizes=(1,),
                      mode=lax.GatherScatterMode.PROMISE_IN_BOUNDS)


def _gather1_kernel(ids1_hbm, emb1_hbm, rows_out_hbm,
                    idx1_v, rows1_v, sem1):
    wid = lax.axis_index("s") * NC + lax.axis_index("c")
    pltpu.sync_copy(ids1_hbm.at[wid], idx1_v)

    def chunk_step(ci, _):
        base = ci * CH

        def issue(g, _):
            vec1 = idx1_v[pl.ds(base + g * 16, 16)]
            for r in range(16):
                pltpu.async_copy(emb1_hbm.at[vec1[r]],
                                 rows1_v.at[g * 16 + r], sem1)
            return 0

        lax.fori_loop(0, CH // 16, issue, 0)

        def drain(j, _):
            pltpu.make_async_copy(emb1_hbm.at[0], rows1_v.at[j], sem1).wait()
            return 0

        lax.fori_loop(0, CH, drain, 0)

        pltpu.sync_copy(rows1_v,
                        rows_out_hbm.at[pl.ds(wid * B_W + base, CH)])
        return 0

    lax.fori_loop(0, NCHUNK, chunk_step, 0)


def _dot2_kernel(ids2_hbm, emb2_hbm, rows1_hbm, out_hbm,
                 idx2_v, rows1_v, rows2_v, out_v, sem1, sem2):
    wid = lax.axis_index("s") * NC + lax.axis_index("c")
    base = wid * B_W

    pltpu.sync_copy(ids2_hbm.at[wid], idx2_v)
    c0 = pltpu.async_copy(rows1_hbm.at[pl.ds(base, B_W)], rows1_v, sem2)
    copies = []
    for c in range(NIC2):
        copies.append(pltpu.async_copy(
            emb2_hbm.at[idx2_v.at[pl.ds(c * ICH, ICH)]],
            rows2_v.at[pl.ds(c * ICH, ICH)], sem1))
    c0.wait()
    for c in copies:
        c.wait()

    lane = lax.iota(jnp.int32, 16)
    perms = [jnp.bitwise_xor(lane, s) for s in (8, 4, 2, 1)]

    def body(g, _):
        outvec = jnp.zeros((16,), jnp.float32)
        for r in range(16):
            j = g * 16 + r
            acc = rows1_v[j, pl.ds(0, 16)] * rows2_v[j, pl.ds(0, 16)]
            for c in range(1, EMB_D // 16):
                acc = acc + (rows1_v[j, pl.ds(c * 16, 16)]
                             * rows2_v[j, pl.ds(c * 16, 16)])
            for p in perms:
                acc = acc + _permute(acc, p)
            outvec = jnp.where(lane == r, acc, outvec)
        out_v[pl.ds(g * 16, 16)] = outvec
        return 0

    lax.fori_loop(0, B_W // 16, body, 0)
    pltpu.sync_copy(out_v, out_hbm.at[wid])


NIC2 = B_W // ICH


@jax.jit
def kernel(ids1, ids2, emb1, emb2):
    mesh = plsc.VectorSubcoreMesh(core_axis_name="c", subcore_axis_name="s",
                                  num_cores=NC, num_subcores=NS)
    g1 = functools.partial(
        pl.kernel,
        out_type=jax.ShapeDtypeStruct((BATCH, EMB_D), jnp.float32),
        mesh=mesh,
        scratch_types=[
            pltpu.VMEM((B_W,), jnp.int32),
            pltpu.VMEM((CH, EMB_D), jnp.float32),
            pltpu.SemaphoreType.DMA,
        ],
    )(_gather1_kernel)
    g2 = functools.partial(
        pl.kernel,
        out_type=jax.ShapeDtypeStruct((NW, B_W), jnp.float32),
        mesh=mesh,
        compiler_params=pltpu.CompilerParams(use_tc_tiling_on_sc=False),
        scratch_types=[
            pltpu.VMEM((B_W,), jnp.int32),
            pltpu.VMEM((B_W, EMB_D), jnp.float32),
            pltpu.VMEM((B_W, EMB_D), jnp.float32),
            pltpu.VMEM((B_W,), jnp.float32),
            pltpu.SemaphoreType.DMA,
            pltpu.SemaphoreType.DMA,
        ],
    )(_dot2_kernel)
    ids1_2d = ids1.astype(jnp.int32).reshape(NW, B_W)
    ids2_2d = ids2.astype(jnp.int32).reshape(NW, B_W)
    rows1 = g1(ids1_2d, emb1)
    out = g2(ids2_2d, emb2, rows1)
    return out.reshape(BATCH, 1)


# speedup vs baseline: 14.3519x; 1.2567x over previous
import functools

import jax
import jax.numpy as jnp
from jax import lax
from jax.experimental import pallas as pl
from jax.experimental.pallas import tpu as pltpu
from jax.experimental.pallas import tpu_sc as plsc

EMB_D = 64
BATCH = 16384
NC = 2
NS = 16
NW = NC * NS
B_W = BATCH // NW
CH = 128
NCHUNK = B_W // CH


def _permute(v, idx16):
    dnums = lax.GatherDimensionNumbers(
        offset_dims=(), collapsed_slice_dims=(0,), start_index_map=(0,))
    return lax.gather(v, idx16[:, None], dnums, slice_sizes=(1,),
                      mode=lax.GatherScatterMode.PROMISE_IN_BOUNDS)


def _mf_kernel(ids1_hbm, ids2_hbm, emb1_hbm, emb2_hbm, out_hbm,
               idx1_v, idx2_v, rows1_v, rows2_v, out_v, sem1, sem2):
    wid = lax.axis_index("s") * NC + lax.axis_index("c")

    pltpu.sync_copy(ids1_hbm.at[wid], idx1_v)
    pltpu.sync_copy(ids2_hbm.at[wid], idx2_v)

    lane = lax.iota(jnp.int32, 16)
    perms = [jnp.bitwise_xor(lane, s) for s in (8, 4, 2, 1)]

    def chunk_step(ci, _):
        base = ci * CH

        def issue(g, _):
            vec1 = idx1_v[pl.ds(base + g * 16, 16)]
            vec2 = idx2_v[pl.ds(base + g * 16, 16)]
            for r in range(16):
                j = g * 16 + r
                pltpu.async_copy(emb1_hbm.at[vec1[r]], rows1_v.at[j], sem1)
                pltpu.async_copy(emb2_hbm.at[vec2[r]], rows2_v.at[j], sem2)
            return 0

        lax.fori_loop(0, CH // 16, issue, 0)

        def drain(j, _):
            pltpu.make_async_copy(emb1_hbm.at[0], rows1_v.at[j], sem1).wait()
            pltpu.make_async_copy(emb2_hbm.at[0], rows2_v.at[j], sem2).wait()
            return 0

        lax.fori_loop(0, CH, drain, 0)

        def body(g, _):
            outvec = jnp.zeros((16,), jnp.float32)
            for r in range(16):
                j = g * 16 + r
                acc = rows1_v[j, pl.ds(0, 16)] * rows2_v[j, pl.ds(0, 16)]
                for c in range(1, EMB_D // 16):
                    acc = acc + (rows1_v[j, pl.ds(c * 16, 16)]
                                 * rows2_v[j, pl.ds(c * 16, 16)])
                for p in perms:
                    acc = acc + _permute(acc, p)
                outvec = jnp.where(lane == r, acc, outvec)
            out_v[pl.ds(base + g * 16, 16)] = outvec
            return 0

        lax.fori_loop(0, CH // 16, body, 0)
        return 0

    lax.fori_loop(0, NCHUNK, chunk_step, 0)

    pltpu.sync_copy(out_v, out_hbm.at[wid])


@jax.jit
def kernel(ids1, ids2, emb1, emb2):
    mesh = plsc.VectorSubcoreMesh(core_axis_name="c", subcore_axis_name="s",
                                  num_cores=NC, num_subcores=NS)
    k = functools.partial(
        pl.kernel,
        out_type=jax.ShapeDtypeStruct((NW, B_W), jnp.float32),
        mesh=mesh,
        scratch_types=[
            pltpu.VMEM((B_W,), jnp.int32),
            pltpu.VMEM((B_W,), jnp.int32),
            pltpu.VMEM((CH, EMB_D), jnp.float32),
            pltpu.VMEM((CH, EMB_D), jnp.float32),
            pltpu.VMEM((B_W,), jnp.float32),
            pltpu.SemaphoreType.DMA,
            pltpu.SemaphoreType.DMA,
        ],
    )(_mf_kernel)
    ids1_2d = ids1.astype(jnp.int32).reshape(NW, B_W)
    ids2_2d = ids2.astype(jnp.int32).reshape(NW, B_W)
    out = k(ids1_2d, ids2_2d, emb1, emb2)
    return out.reshape(BATCH, 1)
